# baseline (speedup 1.0000x reference)
import jax
import jax.numpy as jnp
from jax.experimental import pallas as pl
from jax.experimental.pallas import tpu as pltpu

B, N, L, D = 8, 9, 512, 768
S = N - 1


def _dma_kernel(valid_ref, reps_in, mask_in, reps_out, mask_out,
                zrep, zmask, copy_sem, zero_sem):
    for b in range(B):
        v = valid_ref[b]

        @pl.when(v != 0)
        def _start(b=b):
            pltpu.make_async_copy(
                reps_in.at[b, pl.ds(1, S)], reps_out.at[b], copy_sem).start()
            pltpu.make_async_copy(
                mask_in.at[b, pl.ds(1, S)], mask_out.at[b], copy_sem).start()

        @pl.when(v == 0)
        def _zero(b=b):
            zrep[...] = jnp.zeros_like(zrep)
            zmask[...] = jnp.zeros_like(zmask)
            for s in range(S):
                pltpu.make_async_copy(zrep, reps_out.at[b, s], zero_sem).start()
            pltpu.make_async_copy(zmask, mask_out.at[b], zero_sem).start()

    for b in range(B):
        v = valid_ref[b]

        @pl.when(v != 0)
        def _wait(b=b):
            pltpu.make_async_copy(
                reps_in.at[b, pl.ds(1, S)], reps_out.at[b], copy_sem).wait()
            pltpu.make_async_copy(
                mask_in.at[b, pl.ds(1, S)], mask_out.at[b], copy_sem).wait()

        @pl.when(v == 0)
        def _waitz(b=b):
            for s in range(S):
                pltpu.make_async_copy(zrep, reps_out.at[b, s], zero_sem).wait()
            pltpu.make_async_copy(zmask, mask_out.at[b], zero_sem).wait()


def kernel(token_reps, token_mask, valid_sentences):
    valid_i32 = valid_sentences.astype(jnp.int32)

    reps_out, mask_out = pl.pallas_call(
        _dma_kernel,
        in_specs=[
            pl.BlockSpec(memory_space=pltpu.MemorySpace.SMEM),
            pl.BlockSpec(memory_space=pltpu.MemorySpace.HBM),
            pl.BlockSpec(memory_space=pltpu.MemorySpace.HBM),
        ],
        out_specs=[
            pl.BlockSpec(memory_space=pltpu.MemorySpace.HBM),
            pl.BlockSpec(memory_space=pltpu.MemorySpace.HBM),
        ],
        out_shape=[
            jax.ShapeDtypeStruct((B, S, L, D), jnp.float32),
            jax.ShapeDtypeStruct((B, S, 1, L), jnp.float32),
        ],
        scratch_shapes=[
            pltpu.MemorySpace.VMEM((L, D), jnp.float32),
            pltpu.MemorySpace.VMEM((S, 1, L), jnp.float32),
            pltpu.SemaphoreType.DMA,
            pltpu.SemaphoreType.DMA,
        ],
    )(valid_i32, token_reps, token_mask.reshape(B, N, 1, L))

    return reps_out, mask_out.reshape(B, S, L)


# speedup vs baseline: 44.9403x; 44.9403x over previous
import jax
import jax.numpy as jnp
from jax.experimental import pallas as pl
from jax.experimental.pallas import tpu as pltpu

B, N, L, D = 8, 9, 512, 768
S = N - 1
ROWS_PER_BATCH_IN = N * L
ROWS_PER_BATCH_OUT = S * L
CHUNK = 2048
CPB = ROWS_PER_BATCH_OUT // CHUNK


def _select_kernel(valid_ref, reps_in, mask_in, reps_out, mask_out):
    b = pl.program_id(0)
    v = valid_ref[b]

    @pl.when(v != 0)
    def _copy():
        reps_out[...] = reps_in[...]
        mask_out[...] = mask_in[...]

    @pl.when(v == 0)
    def _zero():
        reps_out[...] = jnp.zeros_like(reps_out)
        mask_out[...] = jnp.zeros_like(mask_out)


def kernel(token_reps, token_mask, valid_sentences):
    valid_i32 = valid_sentences.astype(jnp.int32)
    reps2d = token_reps.reshape(B * N * L, D)
    mask4 = token_mask.reshape(B, N, 1, L)

    reps_out, mask_out = pl.pallas_call(
        _select_kernel,
        grid=(B, CPB),
        in_specs=[
            pl.BlockSpec(memory_space=pltpu.MemorySpace.SMEM),
            pl.BlockSpec(
                (pl.Element(CHUNK), pl.Element(D)),
                lambda b, c: (
                    pl.multiple_of(b * ROWS_PER_BATCH_IN + L + c * CHUNK, 512),
                    0,
                ),
            ),
            pl.BlockSpec(
                (pl.Element(1), pl.Element(S), pl.Element(1), pl.Element(L)),
                lambda b, c: (b, 1, 0, 0),
            ),
        ],
        out_specs=[
            pl.BlockSpec((CHUNK, D), lambda b, c: (b * CPB + c, 0)),
            pl.BlockSpec((1, S, 1, L), lambda b, c: (b, 0, 0, 0)),
        ],
        out_shape=[
            jax.ShapeDtypeStruct((B * S * L, D), jnp.float32),
            jax.ShapeDtypeStruct((B, S, 1, L), jnp.float32),
        ],
    )(valid_i32, reps2d, mask4)

    return reps_out.reshape(B, S, L, D), mask_out.reshape(B, S, L)


# speedup vs baseline: 45.5042x; 1.0125x over previous
import jax
import jax.numpy as jnp
from jax.experimental import pallas as pl
from jax.experimental.pallas import tpu as pltpu

B, N, L, D = 8, 9, 512, 768
S = N - 1
ROWS_PER_BATCH_IN = N * L
ROWS_PER_BATCH_OUT = S * L
CHUNK = 4096
CPB = ROWS_PER_BATCH_OUT // CHUNK


def _select_kernel(valid_ref, reps_in, mask_in, reps_out, mask_out):
    b = pl.program_id(0)
    v = valid_ref[b]

    @pl.when(v != 0)
    def _copy():
        reps_out[...] = reps_in[...]
        mask_out[...] = mask_in[...]

    @pl.when(v == 0)
    def _zero():
        reps_out[...] = jnp.zeros_like(reps_out)
        mask_out[...] = jnp.zeros_like(mask_out)


def kernel(token_reps, token_mask, valid_sentences):
    valid_i32 = valid_sentences.astype(jnp.int32)
    reps2d = token_reps.reshape(B * N * L, D)
    mask4 = token_mask.reshape(B, N, 1, L)

    reps_out, mask_out = pl.pallas_call(
        _select_kernel,
        grid=(B, CPB),
        in_specs=[
            pl.BlockSpec(memory_space=pltpu.MemorySpace.SMEM),
            pl.BlockSpec(
                (pl.Element(CHUNK), pl.Element(D)),
                lambda b, c: (
                    pl.multiple_of(b * ROWS_PER_BATCH_IN + L + c * CHUNK, 512),
                    0,
                ),
            ),
            pl.BlockSpec(
                (pl.Element(1), pl.Element(S), pl.Element(1), pl.Element(L)),
                lambda b, c: (b, 1, 0, 0),
            ),
        ],
        out_specs=[
            pl.BlockSpec((CHUNK, D), lambda b, c: (b * CPB + c, 0)),
            pl.BlockSpec((1, S, 1, L), lambda b, c: (b, 0, 0, 0)),
        ],
        out_shape=[
            jax.ShapeDtypeStruct((B * S * L, D), jnp.float32),
            jax.ShapeDtypeStruct((B, S, 1, L), jnp.float32),
        ],
    )(valid_i32, reps2d, mask4)

    return reps_out.reshape(B, S, L, D), mask_out.reshape(B, S, L)
